# Initial kernel scaffold; baseline (speedup 1.0000x reference)
import jax
import jax.numpy as jnp
from jax.experimental import pallas as pl


def kernel(x, adj_t, W1, b1, g1, be1, W2, b2, g2, be2, W3, b3):
    raise NotImplementedError("write your pallas kernel here")


# speedup vs baseline: 16.2598x; 16.2598x over previous
import functools

import jax
import jax.numpy as jnp
from jax import lax
from jax.experimental import pallas as pl
from jax.experimental.pallas import tpu as pltpu
from jax.experimental.pallas import tpu_sc as plsc

_NC = 2
_NS = 16
_NW = _NC * _NS
_CH = 80


def _make_deg_kernel(np_, nchunk):
  mesh = plsc.VectorSubcoreMesh(core_axis_name="c", subcore_axis_name="s")
  rows_per_tile = np_ // _NS

  @functools.partial(
      pl.kernel,
      out_type=jax.ShapeDtypeStruct((_NC, np_, 16), jnp.float32),
      mesh=mesh,
      scratch_types=[
          pltpu.VMEM((nchunk, _CH), jnp.int32),
          pltpu.VMEM((_CH, 16), jnp.float32),
          pltpu.VMEM_SHARED((np_, 16), jnp.float32),
      ],
  )
  def deg_kernel(dst_hbm, ones_hbm, zeros_hbm, out_hbm, dst_v, ones_v, acc_sh):
    c = lax.axis_index("c")
    s = lax.axis_index("s")
    wid = s * _NC + c
    pltpu.sync_copy(
        zeros_hbm.at[pl.ds(s * rows_per_tile, rows_per_tile)],
        acc_sh.at[pl.ds(s * rows_per_tile, rows_per_tile)],
    )
    pltpu.sync_copy(dst_hbm.at[wid], dst_v)
    pltpu.sync_copy(ones_hbm, ones_v)
    plsc.subcore_barrier()

    def body(j, carry):
      pltpu.sync_copy(ones_v, acc_sh.at[dst_v.at[j]], add=True)
      return carry

    lax.fori_loop(0, nchunk, body, 0)
    plsc.subcore_barrier()
    pltpu.sync_copy(
        acc_sh.at[pl.ds(s * rows_per_tile, rows_per_tile)],
        out_hbm.at[c, pl.ds(s * rows_per_tile, rows_per_tile)],
    )

  return deg_kernel


def _make_agg_kernel(np_, nchunk, d):
  mesh = plsc.VectorSubcoreMesh(core_axis_name="c", subcore_axis_name="s")
  rows_per_tile = np_ // _NS

  @functools.partial(
      pl.kernel,
      out_type=jax.ShapeDtypeStruct((_NC, np_, d), jnp.float32),
      mesh=mesh,
      scratch_types=[
          pltpu.VMEM((nchunk, _CH), jnp.int32),
          pltpu.VMEM((nchunk, _CH), jnp.int32),
          pltpu.VMEM((_CH, d), jnp.float32),
          pltpu.VMEM_SHARED((np_, d), jnp.float32),
          pltpu.SemaphoreType.DMA,
      ],
  )
  def agg_kernel(y_hbm, src_hbm, dst_hbm, zeros_hbm, out_hbm,
                 src_v, dst_v, rows_v, acc_sh, sem):
    c = lax.axis_index("c")
    s = lax.axis_index("s")
    wid = s * _NC + c
    pltpu.sync_copy(
        zeros_hbm.at[pl.ds(s * rows_per_tile, rows_per_tile)],
        acc_sh.at[pl.ds(s * rows_per_tile, rows_per_tile)],
    )
    pltpu.sync_copy(src_hbm.at[wid], src_v)
    pltpu.sync_copy(dst_hbm.at[wid], dst_v)
    plsc.subcore_barrier()

    def body(j, carry):
      pltpu.async_copy(y_hbm.at[src_v.at[j]], rows_v, sem).wait()
      pltpu.sync_copy(rows_v, acc_sh.at[dst_v.at[j]], add=True)
      return carry

    lax.fori_loop(0, nchunk, body, 0)
    plsc.subcore_barrier()
    pltpu.sync_copy(
        acc_sh.at[pl.ds(s * rows_per_tile, rows_per_tile)],
        out_hbm.at[c, pl.ds(s * rows_per_tile, rows_per_tile)],
    )

  return agg_kernel


_BN_RS = 1.0 / (1.0 + 1e-5) ** 0.5


def _first_tc(degp_ref, x_ref, w_ref, dis_ref, y_ref):
  deg = degp_ref[0, :, 0:1] + degp_ref[1, :, 0:1]
  dis = lax.rsqrt(deg)
  dis_ref[...] = dis
  xw = jnp.dot(x_ref[...], w_ref[...], preferred_element_type=jnp.float32)
  y_ref[...] = dis * xw


def _mid_tc(aggp_ref, y_ref, dis_ref, b_ref, g_ref, be_ref, w_ref, out_ref):
  dis = dis_ref[...]
  h = dis * (aggp_ref[0] + aggp_ref[1] + y_ref[...]) + b_ref[...]
  h = h * (g_ref[...] * _BN_RS) + be_ref[...]
  h = jnp.maximum(h, 0.0)
  hw = jnp.dot(h, w_ref[...], preferred_element_type=jnp.float32)
  out_ref[...] = dis * hw


def _last_tc(aggp_ref, y_ref, dis_ref, b_ref, out_ref):
  cdim = out_ref.shape[-1]
  dis = dis_ref[...]
  agg = aggp_ref[0, :, :cdim] + aggp_ref[1, :, :cdim]
  o = dis * (agg + y_ref[:, :cdim]) + b_ref[...]
  m = jnp.max(o, axis=-1, keepdims=True)
  z = o - m
  lse = jnp.log(jnp.sum(jnp.exp(z), axis=-1, keepdims=True))
  out_ref[...] = z - lse


def kernel(x, adj_t, W1, b1, g1, be1, W2, b2, g2, be2, W3, b3):
  n, f_in = x.shape
  e = adj_t.shape[1]
  h = W1.shape[1]
  cdim = W3.shape[1]
  assert e % (_NW * _CH) == 0
  nchunk = e // (_NW * _CH)
  np_ = ((n + 8 * _NS - 1) // (8 * _NS)) * (8 * _NS)

  src = adj_t[0].reshape(_NW, nchunk, _CH)
  dst = adj_t[1].reshape(_NW, nchunk, _CH)

  ones16 = jnp.ones((_CH, 16), jnp.float32)
  zeros16 = jnp.zeros((np_, 16), jnp.float32)
  zeros_h = jnp.zeros((np_, h), jnp.float32)
  W3p = jnp.pad(W3, ((0, 0), (0, h - cdim)))

  degp = _make_deg_kernel(np_, nchunk)(dst, ones16, zeros16)

  blk = 1000
  assert n % blk == 0
  grid = (n // blk,)

  def full(shape):
    return pl.BlockSpec(shape, lambda i: tuple(0 for _ in shape))

  def rows2(d):
    return pl.BlockSpec((blk, d), lambda i: (i, 0))

  def parts(d):
    return pl.BlockSpec((_NC, blk, d), lambda i: (0, i, 0))

  dis, y1 = pl.pallas_call(
      _first_tc,
      grid=grid,
          in_specs=[parts(16), rows2(f_in), full((f_in, h))],
      out_specs=[rows2(1), rows2(h)],
      out_shape=[
          jax.ShapeDtypeStruct((n, 1), jnp.float32),
          jax.ShapeDtypeStruct((n, h), jnp.float32),
      ],
  )(degp, x, W1)

  agg_h = _make_agg_kernel(np_, nchunk, h)
  b1r, g1r, be1r = b1.reshape(1, h), g1.reshape(1, h), be1.reshape(1, h)
  b2r, g2r, be2r = b2.reshape(1, h), g2.reshape(1, h), be2.reshape(1, h)
  b3r = b3.reshape(1, cdim)

  agg1 = agg_h(y1, src, dst, zeros_h)
  y2 = pl.pallas_call(
      _mid_tc,
      grid=grid,
      in_specs=[parts(h), rows2(h), rows2(1), full((1, h)), full((1, h)),
                full((1, h)), full((h, h))],
      out_specs=rows2(h),
      out_shape=jax.ShapeDtypeStruct((n, h), jnp.float32),
  )(agg1, y1, dis, b1r, g1r, be1r, W2)

  agg2 = agg_h(y2, src, dst, zeros_h)
  y3 = pl.pallas_call(
      _mid_tc,
      grid=grid,
      in_specs=[parts(h), rows2(h), rows2(1), full((1, h)), full((1, h)),
                full((1, h)), full((h, h))],
      out_specs=rows2(h),
      out_shape=jax.ShapeDtypeStruct((n, h), jnp.float32),
  )(agg2, y2, dis, b2r, g2r, be2r, W3p)

  agg3 = agg_h(y3, src, dst, zeros_h)
  out = pl.pallas_call(
      _last_tc,
      grid=grid,
      in_specs=[parts(h), rows2(h), rows2(1), full((1, cdim))],
      out_specs=rows2(cdim),
      out_shape=jax.ShapeDtypeStruct((n, cdim), jnp.float32),
  )(agg3, y3, dis, b3r)
  return out


# speedup vs baseline: 26.8791x; 1.6531x over previous
import functools

import jax
import jax.numpy as jnp
from jax import lax
from jax.experimental import pallas as pl
from jax.experimental.pallas import tpu as pltpu
from jax.experimental.pallas import tpu_sc as plsc

_NC = 2
_NS = 16
_NW = _NC * _NS
_CH = 125


def _make_deg_kernel(np_, nchunk):
  mesh = plsc.VectorSubcoreMesh(core_axis_name="c", subcore_axis_name="s")
  rows_per_tile = np_ // _NS

  @functools.partial(
      pl.kernel,
      out_type=jax.ShapeDtypeStruct((_NC, np_, 16), jnp.float32),
      mesh=mesh,
      scratch_types=[
          pltpu.VMEM((nchunk, _CH), jnp.int32),
          pltpu.VMEM((_CH, 16), jnp.float32),
          pltpu.VMEM_SHARED((np_, 16), jnp.float32),
      ],
  )
  def deg_kernel(dst_hbm, ones_hbm, zeros_hbm, out_hbm, dst_v, ones_v, acc_sh):
    c = lax.axis_index("c")
    s = lax.axis_index("s")
    wid = s * _NC + c
    pltpu.sync_copy(
        zeros_hbm.at[pl.ds(s * rows_per_tile, rows_per_tile)],
        acc_sh.at[pl.ds(s * rows_per_tile, rows_per_tile)],
    )
    pltpu.sync_copy(dst_hbm.at[wid], dst_v)
    pltpu.sync_copy(ones_hbm, ones_v)
    plsc.subcore_barrier()

    def body(j, carry):
      pltpu.sync_copy(ones_v, acc_sh.at[dst_v.at[j]], add=True)
      return carry

    lax.fori_loop(0, nchunk, body, 0)
    plsc.subcore_barrier()
    pltpu.sync_copy(
        acc_sh.at[pl.ds(s * rows_per_tile, rows_per_tile)],
        out_hbm.at[c, pl.ds(s * rows_per_tile, rows_per_tile)],
    )

  return deg_kernel


def _make_agg_kernel(np_, nchunk, d):
  mesh = plsc.VectorSubcoreMesh(core_axis_name="c", subcore_axis_name="s")
  rows_per_tile = np_ // _NS

  @functools.partial(
      pl.kernel,
      out_type=jax.ShapeDtypeStruct((_NC, np_, d), jnp.float32),
      mesh=mesh,
      scratch_types=[
          pltpu.VMEM((nchunk // 2, _CH), jnp.int32),
          pltpu.VMEM((nchunk // 2, _CH), jnp.int32),
          pltpu.VMEM((_CH, d), jnp.float32),
          pltpu.VMEM((_CH, d), jnp.float32),
          pltpu.VMEM_SHARED((np_, d), jnp.float32),
          pltpu.SemaphoreType.DMA,
          pltpu.SemaphoreType.DMA,
      ],
  )
  def agg_kernel(y_hbm, src_hbm, dst_hbm, zeros_hbm, out_hbm,
                 src_v, dst_v, rows_a, rows_b, acc_sh, sem_a, sem_b):
    c = lax.axis_index("c")
    s = lax.axis_index("s")
    wid = s * _NC + c
    pltpu.sync_copy(
        zeros_hbm.at[pl.ds(s * rows_per_tile, rows_per_tile)],
        acc_sh.at[pl.ds(s * rows_per_tile, rows_per_tile)],
    )
    plsc.subcore_barrier()

    hc = nchunk // 2
    for half in range(2):
      pltpu.sync_copy(src_hbm.at[wid, pl.ds(half * hc, hc)], src_v)
      pltpu.sync_copy(dst_hbm.at[wid, pl.ds(half * hc, hc)], dst_v)
      pltpu.async_copy(y_hbm.at[src_v.at[0]], rows_a, sem_a)

      def body(k, carry):
        j = 2 * k
        pltpu.async_copy(y_hbm.at[src_v.at[j + 1]], rows_b, sem_b)
        pltpu.make_async_copy(y_hbm.at[src_v.at[j]], rows_a, sem_a).wait()
        pltpu.sync_copy(rows_a, acc_sh.at[dst_v.at[j]], add=True)

        @pl.when(k < hc // 2 - 1)
        def _():
          pltpu.async_copy(y_hbm.at[src_v.at[j + 2]], rows_a, sem_a)

        pltpu.make_async_copy(y_hbm.at[src_v.at[j + 1]], rows_b, sem_b).wait()
        pltpu.sync_copy(rows_b, acc_sh.at[dst_v.at[j + 1]], add=True)
        return carry

      lax.fori_loop(0, hc // 2, body, 0)
    plsc.subcore_barrier()
    pltpu.sync_copy(
        acc_sh.at[pl.ds(s * rows_per_tile, rows_per_tile)],
        out_hbm.at[c, pl.ds(s * rows_per_tile, rows_per_tile)],
    )

  return agg_kernel


_BN_RS = 1.0 / (1.0 + 1e-5) ** 0.5


def _first_tc(degp_ref, x_ref, w_ref, dis_ref, y_ref):
  deg = degp_ref[0, :, 0:1] + degp_ref[1, :, 0:1]
  dis = lax.rsqrt(deg)
  dis_ref[...] = dis
  xw = jnp.dot(x_ref[...], w_ref[...], preferred_element_type=jnp.float32)
  y_ref[...] = dis * xw


def _mid_tc(aggp_ref, y_ref, dis_ref, b_ref, g_ref, be_ref, w_ref, out_ref):
  dis = dis_ref[...]
  h = dis * (aggp_ref[0] + aggp_ref[1] + y_ref[...]) + b_ref[...]
  h = h * (g_ref[...] * _BN_RS) + be_ref[...]
  h = jnp.maximum(h, 0.0)
  hw = jnp.dot(h, w_ref[...], preferred_element_type=jnp.float32)
  out_ref[...] = dis * hw


def _last_tc(aggp_ref, y_ref, dis_ref, b_ref, out_ref):
  cdim = out_ref.shape[-1]
  dis = dis_ref[...]
  agg = aggp_ref[0, :, :cdim] + aggp_ref[1, :, :cdim]
  o = dis * (agg + y_ref[:, :cdim]) + b_ref[...]
  m = jnp.max(o, axis=-1, keepdims=True)
  z = o - m
  lse = jnp.log(jnp.sum(jnp.exp(z), axis=-1, keepdims=True))
  out_ref[...] = z - lse


def kernel(x, adj_t, W1, b1, g1, be1, W2, b2, g2, be2, W3, b3):
  n, f_in = x.shape
  e = adj_t.shape[1]
  h = W1.shape[1]
  cdim = W3.shape[1]
  assert e % (_NW * _CH) == 0
  nchunk = e // (_NW * _CH)
  assert nchunk % 4 == 0
  np_ = ((n + 8 * _NS - 1) // (8 * _NS)) * (8 * _NS)

  src = adj_t[0].reshape(_NW, nchunk, _CH)
  dst = adj_t[1].reshape(_NW, nchunk, _CH)

  ones16 = jnp.ones((_CH, 16), jnp.float32)
  zeros16 = jnp.zeros((np_, 16), jnp.float32)
  zeros_h = jnp.zeros((np_, h), jnp.float32)
  W3p = jnp.pad(W3, ((0, 0), (0, h - cdim)))

  degp = _make_deg_kernel(np_, nchunk)(dst, ones16, zeros16)

  blk = 1000
  assert n % blk == 0
  grid = (n // blk,)

  def full(shape):
    return pl.BlockSpec(shape, lambda i: tuple(0 for _ in shape))

  def rows2(d):
    return pl.BlockSpec((blk, d), lambda i: (i, 0))

  def parts(d):
    return pl.BlockSpec((_NC, blk, d), lambda i: (0, i, 0))

  dis, y1 = pl.pallas_call(
      _first_tc,
      grid=grid,
          in_specs=[parts(16), rows2(f_in), full((f_in, h))],
      out_specs=[rows2(1), rows2(h)],
      out_shape=[
          jax.ShapeDtypeStruct((n, 1), jnp.float32),
          jax.ShapeDtypeStruct((n, h), jnp.float32),
      ],
  )(degp, x, W1)

  agg_h = _make_agg_kernel(np_, nchunk, h)
  b1r, g1r, be1r = b1.reshape(1, h), g1.reshape(1, h), be1.reshape(1, h)
  b2r, g2r, be2r = b2.reshape(1, h), g2.reshape(1, h), be2.reshape(1, h)
  b3r = b3.reshape(1, cdim)

  agg1 = agg_h(y1, src, dst, zeros_h)
  y2 = pl.pallas_call(
      _mid_tc,
      grid=grid,
      in_specs=[parts(h), rows2(h), rows2(1), full((1, h)), full((1, h)),
                full((1, h)), full((h, h))],
      out_specs=rows2(h),
      out_shape=jax.ShapeDtypeStruct((n, h), jnp.float32),
  )(agg1, y1, dis, b1r, g1r, be1r, W2)

  agg2 = agg_h(y2, src, dst, zeros_h)
  y3 = pl.pallas_call(
      _mid_tc,
      grid=grid,
      in_specs=[parts(h), rows2(h), rows2(1), full((1, h)), full((1, h)),
                full((1, h)), full((h, h))],
      out_specs=rows2(h),
      out_shape=jax.ShapeDtypeStruct((n, h), jnp.float32),
  )(agg2, y2, dis, b2r, g2r, be2r, W3p)

  agg3 = agg_h(y3, src, dst, zeros_h)
  out = pl.pallas_call(
      _last_tc,
      grid=grid,
      in_specs=[parts(h), rows2(h), rows2(1), full((1, cdim))],
      out_specs=rows2(cdim),
      out_shape=jax.ShapeDtypeStruct((n, cdim), jnp.float32),
  )(agg3, y3, dis, b3r)
  return out
